# baseline (speedup 1.0000x reference)
import functools

import jax
import jax.numpy as jnp
from jax.experimental import pallas as pl
from jax.experimental.pallas import tpu as pltpu
from jax.experimental.pallas import tpu_sc as plsc

_R = 40
_NUM_SUBCORES = 32


def _sc_sum(x):
    K, E, d = x.shape
    blocks = E // _R
    per = blocks // _NUM_SUBCORES
    mesh = plsc.VectorSubcoreMesh(core_axis_name="c", subcore_axis_name="s")

    @functools.partial(
        pl.kernel,
        out_type=jax.ShapeDtypeStruct((E, d), x.dtype),
        mesh=mesh,
    )
    def sc_sum_kernel(x_hbm, o_hbm):
        def body(a_ref, b_ref, c_ref, d_ref, o_ref):
            a2, b2, c2, d2 = a_ref.at[0], b_ref.at[0], c_ref.at[0], d_ref.at[0]

            @pl.loop(0, _R)
            def _(r):
                @pl.loop(0, d, step=16)
                def _(c):
                    s = pl.ds(c, 16)
                    o_ref[r, s] = a2[r, s] + b2[r, s] + c2[r, s] + d2[r, s]

        pltpu.emit_pipeline(
            body,
            grid=(_NUM_SUBCORES, per),
            in_specs=[
                pl.BlockSpec((1, _R, d), index_map=lambda i, j, k=k: (k, i * per + j, 0))
                for k in range(K)
            ],
            out_specs=[
                pl.BlockSpec((_R, d), index_map=lambda i, j: (i * per + j, 0))
            ],
            core_axis_name=("c", "s"),
            dimension_semantics=(pltpu.PARALLEL, pltpu.ARBITRARY),
        )(x_hbm, x_hbm, x_hbm, x_hbm, o_hbm)

    return sc_sum_kernel(x)


def kernel(encoded_edges, encoded_history):
    return _sc_sum(encoded_edges)


# speedup vs baseline: 2.0981x; 2.0981x over previous
import functools

import jax
import jax.numpy as jnp
from jax.experimental import pallas as pl
from jax.experimental.pallas import tpu as pltpu
from jax.experimental.pallas import tpu_sc as plsc

_R = 40
_NUM_SUBCORES = 32
_E_SC = 64000
_BR = 8000


def _sc_sum_tail(x, e_tc):
    K, E, d = x.shape
    e_sc = E - e_tc
    per = (e_sc // _R) // _NUM_SUBCORES
    row0 = e_tc // _R
    mesh = plsc.VectorSubcoreMesh(core_axis_name="c", subcore_axis_name="s")

    @functools.partial(
        pl.kernel,
        out_type=jax.ShapeDtypeStruct((e_sc, d), x.dtype),
        mesh=mesh,
    )
    def sc_sum_kernel(x_hbm, o_hbm):
        def body(a_ref, b_ref, c_ref, d_ref, o_ref):
            a2, b2, c2, d2 = a_ref.at[0], b_ref.at[0], c_ref.at[0], d_ref.at[0]

            @pl.loop(0, _R)
            def _(r):
                @pl.loop(0, d, step=16)
                def _(c):
                    s = pl.ds(c, 16)
                    o_ref[r, s] = a2[r, s] + b2[r, s] + c2[r, s] + d2[r, s]

        pltpu.emit_pipeline(
            body,
            grid=(_NUM_SUBCORES, per),
            in_specs=[
                pl.BlockSpec(
                    (1, _R, d),
                    index_map=lambda i, j, k=k: (k, row0 + i * per + j, 0),
                )
                for k in range(K)
            ],
            out_specs=[
                pl.BlockSpec((_R, d), index_map=lambda i, j: (i * per + j, 0))
            ],
            core_axis_name=("c", "s"),
            dimension_semantics=(pltpu.PARALLEL, pltpu.ARBITRARY),
        )(x_hbm, x_hbm, x_hbm, x_hbm, o_hbm)

    return sc_sum_kernel(x)


def _tc_sum_kernel(x_ref, o_ref):
    x = x_ref[...]
    o_ref[...] = x[0] + x[1] + x[2] + x[3]


def _tc_sum_head(x, e_tc):
    K, E, d = x.shape
    return pl.pallas_call(
        _tc_sum_kernel,
        grid=(e_tc // _BR,),
        in_specs=[pl.BlockSpec((K, _BR, d), lambda i: (0, i, 0))],
        out_specs=pl.BlockSpec((_BR, d), lambda i: (i, 0)),
        out_shape=jax.ShapeDtypeStruct((E, d), x.dtype),
    )(x)


def kernel(encoded_edges, encoded_history):
    K, E, d = encoded_edges.shape
    e_tc = E - _E_SC
    sc_out = _sc_sum_tail(encoded_edges, e_tc)
    tc_out = _tc_sum_head(encoded_edges, e_tc)
    return jax.lax.dynamic_update_slice(tc_out, sc_out, (e_tc, 0))


# speedup vs baseline: 2.2639x; 1.0790x over previous
import functools

import jax
import jax.numpy as jnp
from jax.experimental import pallas as pl
from jax.experimental.pallas import tpu as pltpu
from jax.experimental.pallas import tpu_sc as plsc

_R = 40
_NUM_SUBCORES = 32
_E_SC = 64000
_BR = 8000


def _sc_sum_tail(x, e_tc):
    K, E, d = x.shape
    e_sc = E - e_tc
    per = (e_sc // _R) // _NUM_SUBCORES
    row0 = e_tc // _R
    mesh = plsc.VectorSubcoreMesh(core_axis_name="c", subcore_axis_name="s")

    @functools.partial(
        pl.kernel,
        out_type=jax.ShapeDtypeStruct((e_sc, d), x.dtype),
        mesh=mesh,
    )
    def sc_sum_kernel(x_hbm, o_hbm):
        def body(a_ref, b_ref, c_ref, d_ref, o_ref):
            a2, b2, c2, d2 = a_ref.at[0], b_ref.at[0], c_ref.at[0], d_ref.at[0]

            @pl.loop(0, _R)
            def _(r):
                @pl.loop(0, d, step=16)
                def _(c):
                    s = pl.ds(c, 16)
                    o_ref[r, s] = a2[r, s] + b2[r, s] + c2[r, s] + d2[r, s]

        pltpu.emit_pipeline(
            body,
            grid=(_NUM_SUBCORES, per),
            in_specs=[
                pl.BlockSpec(
                    (1, _R, d),
                    index_map=lambda i, j, k=k: (k, row0 + i * per + j, 0),
                )
                for k in range(K)
            ],
            out_specs=[
                pl.BlockSpec((_R, d), index_map=lambda i, j: (i * per + j, 0))
            ],
            core_axis_name=("c", "s"),
            dimension_semantics=(pltpu.PARALLEL, pltpu.ARBITRARY),
        )(x_hbm, x_hbm, x_hbm, x_hbm, o_hbm)

    return sc_sum_kernel(x)


def _tc_sum_kernel(x_ref, o_ref):
    x = x_ref[...]
    o_ref[...] = x[0] + x[1] + x[2] + x[3]


def _tc_sum_head(x, e_tc):
    K, E, d = x.shape
    return pl.pallas_call(
        _tc_sum_kernel,
        grid=(e_tc // _BR,),
        in_specs=[pl.BlockSpec((K, _BR, d), lambda i: (0, i, 0))],
        out_specs=pl.BlockSpec((_BR, d), lambda i: (i, 0)),
        out_shape=jax.ShapeDtypeStruct((E, d), x.dtype),
    )(x)


def kernel(encoded_edges, encoded_history):
    K, E, d = encoded_edges.shape
    e_tc = E - _E_SC
    sc_out = _sc_sum_tail(encoded_edges, e_tc)
    tc_out = _tc_sum_head(encoded_edges, e_tc)
    return (tc_out, sc_out)


# speedup vs baseline: 2.4785x; 1.0948x over previous
import jax
import jax.numpy as jnp
from jax.experimental import pallas as pl


def _sum_k_kernel(x_ref, o_ref):
    x = x_ref[...]
    o_ref[...] = x[0] + x[1] + x[2] + x[3]


def kernel(encoded_edges, encoded_history):
    K, E, d = encoded_edges.shape
    BR = 6400
    grid = (E // BR,)
    return pl.pallas_call(
        _sum_k_kernel,
        grid=grid,
        in_specs=[pl.BlockSpec((K, BR, d), lambda i: (0, i, 0))],
        out_specs=pl.BlockSpec((BR, d), lambda i: (i, 0)),
        out_shape=jax.ShapeDtypeStruct((E, d), encoded_edges.dtype),
    )(encoded_edges)
